# baseline (speedup 1.0000x reference)
import functools

import jax
import jax.numpy as jnp
from jax import lax
from jax.experimental import pallas as pl
from jax.experimental.pallas import tpu as pltpu
from jax.experimental.pallas import tpu_sc as plsc


def _degree_body(vmax, x_ref, deg_ref, acc_ref):
    i = pl.program_id(1)

    @pl.when(i == 0)
    def _init():
        acc_ref[...] = jnp.zeros_like(acc_ref)

    acc_ref[...] += jnp.sum(x_ref[...], axis=0, keepdims=True)

    @pl.when(i == pl.num_programs(1) - 1)
    def _fin():
        deg = jnp.ceil(acc_ref[...]).astype(jnp.int32)
        deg_ref[...] = jnp.clip(deg, 0, vmax)


def _degree(x, vmax, col_off, col_width, block_cols=2560, block_rows=200):
    n_rows, _ = x.shape
    cb0 = col_off // block_cols
    n_cb = pl.cdiv(col_width, block_cols)
    grid = (n_cb, pl.cdiv(n_rows, block_rows))
    out = pl.pallas_call(
        functools.partial(_degree_body, vmax),
        grid=grid,
        in_specs=[pl.BlockSpec((block_rows, block_cols),
                               lambda j, i: (i, cb0 + j))],
        out_specs=pl.BlockSpec((1, block_cols), lambda j, i: (0, j)),
        out_shape=jax.ShapeDtypeStruct((1, col_width), jnp.int32),
        scratch_shapes=[pltpu.VMEM((1, block_cols), jnp.float32)],
    )(x)
    return out.reshape(col_width)


_LANES = 16


def _gather_add(out_r, W, feat, deg, row_base, count, b_per_w, chunk=80):
    _, D = feat.shape
    mesh = plsc.VectorSubcoreMesh(core_axis_name="c", subcore_axis_name="s")
    n_chunks = b_per_w // chunk
    vecs_per_row = D // _LANES

    @functools.partial(
        pl.kernel,
        mesh=mesh,
        out_type=(),
        scratch_types=[
            pltpu.VMEM((b_per_w,), jnp.int32),
            pltpu.VMEM((chunk, D), jnp.float32),
            pltpu.VMEM((chunk, D), jnp.float32),
            pltpu.VMEM((chunk, D), jnp.float32),
            pltpu.VMEM((chunk, D), jnp.float32),
            pltpu.SemaphoreType.DMA,
            pltpu.SemaphoreType.DMA,
            pltpu.SemaphoreType.DMA,
            pltpu.SemaphoreType.DMA,
            pltpu.SemaphoreType.DMA,
            pltpu.SemaphoreType.DMA,
        ],
    )
    def k(w_hbm, feat_hbm, deg_hbm, out_hbm,
          idx_v, rows0, rows1, featb0, featb1,
          gsem0, gsem1, fsem0, fsem1, osem0, osem1):
        rows = (rows0, rows1)
        featb = (featb0, featb1)
        gsem = (gsem0, gsem1)
        fsem = (fsem0, fsem1)
        osem = (osem0, osem1)
        wid = lax.axis_index("s") * 2 + lax.axis_index("c")
        local = jnp.minimum(wid * b_per_w, count - b_per_w)
        base = row_base + local

        grs = [None, None]
        frs = [None, None]
        ows = [None, None]

        def start_feat(c):
            buf = c % 2
            frs[buf] = pltpu.async_copy(
                feat_hbm.at[pl.ds(base + c * chunk, chunk)], featb[buf], fsem[buf])

        def start_gather(c):
            buf = c % 2
            grs[buf] = pltpu.async_copy(
                w_hbm.at[idx_v.at[pl.ds(c * chunk, chunk)]], rows[buf], gsem[buf])

        def start_in(c):
            start_gather(c)
            start_feat(c)

        start_feat(0)
        pltpu.sync_copy(deg_hbm.at[pl.ds(local, b_per_w)], idx_v)
        start_gather(0)
        for c in range(n_chunks):
            buf = c % 2
            if c + 1 < n_chunks:
                nxt = (c + 1) % 2
                if ows[nxt] is not None:
                    ows[nxt].wait()
                    ows[nxt] = None
                start_in(c + 1)
            grs[buf].wait()
            frs[buf].wait()

            rows_ref = rows[buf]
            featb_ref = featb[buf]

            @plsc.parallel_loop(0, chunk, 1, unroll=4)
            def _row(j):
                for kk in range(vecs_per_row):
                    sl = pl.ds(kk * _LANES, _LANES)
                    plsc.addupdate(featb_ref.at[j, sl], rows_ref[j, sl])

            ows[buf] = pltpu.async_copy(
                featb[buf], out_hbm.at[pl.ds(base + c * chunk, chunk)], osem[buf])
        for d in ows:
            if d is not None:
                d.wait()

    k(W, feat, deg, out_r)


_SPLIT = 7680


def kernel(x, feat, W):
    n = x.shape[1]
    vmax = W.shape[0] - 1
    deg_a = _degree(x, vmax, 0, _SPLIT)
    deg_b = _degree(x, vmax, _SPLIT, n - _SPLIT)
    out_r = jax.new_ref(jnp.zeros(feat.shape, feat.dtype))
    _gather_add(out_r, W, feat, deg_a, 0, _SPLIT, b_per_w=_SPLIT // 32)
    _gather_add(out_r, W, feat, deg_b, _SPLIT, n - _SPLIT, b_per_w=80)
    return out_r[...]


# speedup vs baseline: 1.2654x; 1.2654x over previous
import functools

import jax
import jax.numpy as jnp
from jax import lax
from jax.experimental import pallas as pl
from jax.experimental.pallas import tpu as pltpu
from jax.experimental.pallas import tpu_sc as plsc


def _degree_body(vmax, x_ref, deg_ref, acc_ref):
    i = pl.program_id(1)

    @pl.when(i == 0)
    def _init():
        acc_ref[...] = jnp.zeros_like(acc_ref)

    acc_ref[...] += jnp.sum(x_ref[...], axis=0, keepdims=True)

    @pl.when(i == pl.num_programs(1) - 1)
    def _fin():
        deg = jnp.ceil(acc_ref[...]).astype(jnp.int32)
        deg_ref[...] = jnp.clip(deg, 0, vmax)


def _degree(x, vmax, col_off, col_width, block_cols=2560, block_rows=200):
    n_rows, _ = x.shape
    cb0 = col_off // block_cols
    n_cb = pl.cdiv(col_width, block_cols)
    grid = (n_cb, pl.cdiv(n_rows, block_rows))
    out = pl.pallas_call(
        functools.partial(_degree_body, vmax),
        grid=grid,
        in_specs=[pl.BlockSpec((block_rows, block_cols),
                               lambda j, i: (i, cb0 + j))],
        out_specs=pl.BlockSpec((1, block_cols), lambda j, i: (0, j)),
        out_shape=jax.ShapeDtypeStruct((1, col_width), jnp.int32),
        scratch_shapes=[pltpu.VMEM((1, block_cols), jnp.float32)],
    )(x)
    return out.reshape(col_width)


_LANES = 16


def _gather_add(out_r, W, feat, deg, row_base, count, b_per_w, chunk=80):
    _, D = feat.shape
    mesh = plsc.VectorSubcoreMesh(core_axis_name="c", subcore_axis_name="s")
    n_chunks = b_per_w // chunk
    vecs_per_row = D // _LANES

    @functools.partial(
        pl.kernel,
        mesh=mesh,
        out_type=(),
        scratch_types=[
            pltpu.VMEM((b_per_w,), jnp.int32),
            pltpu.VMEM((chunk, D), jnp.float32),
            pltpu.VMEM((chunk, D), jnp.float32),
            pltpu.VMEM((chunk, D), jnp.float32),
            pltpu.VMEM((chunk, D), jnp.float32),
            pltpu.SemaphoreType.DMA,
            pltpu.SemaphoreType.DMA,
            pltpu.SemaphoreType.DMA,
            pltpu.SemaphoreType.DMA,
            pltpu.SemaphoreType.DMA,
            pltpu.SemaphoreType.DMA,
        ],
    )
    def k(w_hbm, feat_hbm, deg_hbm, out_hbm,
          idx_v, rows0, rows1, featb0, featb1,
          gsem0, gsem1, fsem0, fsem1, osem0, osem1):
        rows = (rows0, rows1)
        featb = (featb0, featb1)
        gsem = (gsem0, gsem1)
        fsem = (fsem0, fsem1)
        osem = (osem0, osem1)
        wid = lax.axis_index("s") * 2 + lax.axis_index("c")
        local = jnp.minimum(wid * b_per_w, count - b_per_w)
        base = row_base + local

        grs = [None, None]
        frs = [None, None]
        ows = [None, None]

        def start_feat(c):
            buf = c % 2
            frs[buf] = pltpu.async_copy(
                feat_hbm.at[pl.ds(base + c * chunk, chunk)], featb[buf], fsem[buf])

        def start_gather(c):
            buf = c % 2
            grs[buf] = pltpu.async_copy(
                w_hbm.at[idx_v.at[pl.ds(c * chunk, chunk)]], rows[buf], gsem[buf])

        def start_in(c):
            start_gather(c)
            start_feat(c)

        start_feat(0)
        pltpu.sync_copy(deg_hbm.at[pl.ds(local, b_per_w)], idx_v)
        start_gather(0)
        for c in range(n_chunks):
            buf = c % 2
            if c + 1 < n_chunks:
                nxt = (c + 1) % 2
                if ows[nxt] is not None:
                    ows[nxt].wait()
                    ows[nxt] = None
                start_in(c + 1)
            grs[buf].wait()
            frs[buf].wait()

            rows_ref = rows[buf]
            featb_ref = featb[buf]

            @plsc.parallel_loop(0, chunk, 1, unroll=4)
            def _row(j):
                for kk in range(vecs_per_row):
                    sl = pl.ds(kk * _LANES, _LANES)
                    plsc.addupdate(featb_ref.at[j, sl], rows_ref[j, sl])

            ows[buf] = pltpu.async_copy(
                featb[buf], out_hbm.at[pl.ds(base + c * chunk, chunk)], osem[buf])
        for d in ows:
            if d is not None:
                d.wait()

    k(W, feat, deg, out_r)


_SPLIT = 7680


def kernel(x, feat, W):
    n = x.shape[1]
    vmax = W.shape[0] - 1
    deg_a = _degree(x, vmax, 0, _SPLIT, block_cols=_SPLIT)
    deg_b = _degree(x, vmax, _SPLIT, n - _SPLIT)
    out_r = jax.new_ref(jnp.zeros(feat.shape, feat.dtype))
    _gather_add(out_r, W, feat, deg_a, 0, _SPLIT, b_per_w=_SPLIT // 32)
    _gather_add(out_r, W, feat, deg_b, _SPLIT, n - _SPLIT, b_per_w=80)
    return out_r[...]


# speedup vs baseline: 1.2683x; 1.0023x over previous
import functools

import jax
import jax.numpy as jnp
from jax import lax
from jax.experimental import pallas as pl
from jax.experimental.pallas import tpu as pltpu
from jax.experimental.pallas import tpu_sc as plsc


def _degree_body(vmax, x_ref, deg_ref, acc_ref):
    i = pl.program_id(1)

    @pl.when(i == 0)
    def _init():
        acc_ref[...] = jnp.zeros_like(acc_ref)

    acc_ref[...] += jnp.sum(x_ref[...], axis=0, keepdims=True)

    @pl.when(i == pl.num_programs(1) - 1)
    def _fin():
        deg = jnp.ceil(acc_ref[...]).astype(jnp.int32)
        deg_ref[...] = jnp.clip(deg, 0, vmax)


def _degree(x, vmax, col_off, col_width, block_cols=2560, block_rows=200):
    n_rows, _ = x.shape
    cb0 = col_off // block_cols
    n_cb = pl.cdiv(col_width, block_cols)
    grid = (n_cb, pl.cdiv(n_rows, block_rows))
    out = pl.pallas_call(
        functools.partial(_degree_body, vmax),
        grid=grid,
        in_specs=[pl.BlockSpec((block_rows, block_cols),
                               lambda j, i: (i, cb0 + j))],
        out_specs=pl.BlockSpec((1, block_cols), lambda j, i: (0, j)),
        out_shape=jax.ShapeDtypeStruct((1, col_width), jnp.int32),
        scratch_shapes=[pltpu.VMEM((1, block_cols), jnp.float32)],
    )(x)
    return out.reshape(col_width)


_LANES = 16


def _gather_add(out_r, W, feat, deg, row_base, count, b_per_w, chunk=80):
    _, D = feat.shape
    mesh = plsc.VectorSubcoreMesh(core_axis_name="c", subcore_axis_name="s")
    n_chunks = b_per_w // chunk
    vecs_per_row = D // _LANES

    @functools.partial(
        pl.kernel,
        mesh=mesh,
        out_type=(),
        scratch_types=[
            pltpu.VMEM((b_per_w,), jnp.int32),
            pltpu.VMEM((chunk, D), jnp.float32),
            pltpu.VMEM((chunk, D), jnp.float32),
            pltpu.VMEM((chunk, D), jnp.float32),
            pltpu.VMEM((chunk, D), jnp.float32),
            pltpu.SemaphoreType.DMA,
            pltpu.SemaphoreType.DMA,
            pltpu.SemaphoreType.DMA,
            pltpu.SemaphoreType.DMA,
            pltpu.SemaphoreType.DMA,
            pltpu.SemaphoreType.DMA,
        ],
    )
    def k(w_hbm, feat_hbm, deg_hbm, out_hbm,
          idx_v, rows0, rows1, featb0, featb1,
          gsem0, gsem1, fsem0, fsem1, osem0, osem1):
        rows = (rows0, rows1)
        featb = (featb0, featb1)
        gsem = (gsem0, gsem1)
        fsem = (fsem0, fsem1)
        osem = (osem0, osem1)
        wid = lax.axis_index("s") * 2 + lax.axis_index("c")
        local = jnp.minimum(wid * b_per_w, count - b_per_w)
        base = row_base + local

        grs = [None, None]
        frs = [None, None]
        ows = [None, None]

        def start_feat(c):
            buf = c % 2
            frs[buf] = pltpu.async_copy(
                feat_hbm.at[pl.ds(base + c * chunk, chunk)], featb[buf], fsem[buf])

        def start_gather(c):
            buf = c % 2
            grs[buf] = pltpu.async_copy(
                w_hbm.at[idx_v.at[pl.ds(c * chunk, chunk)]], rows[buf], gsem[buf])

        def start_in(c):
            start_gather(c)
            start_feat(c)

        start_feat(0)
        pltpu.sync_copy(deg_hbm.at[pl.ds(local, b_per_w)], idx_v)
        start_gather(0)
        for c in range(n_chunks):
            buf = c % 2
            if c + 1 < n_chunks:
                nxt = (c + 1) % 2
                if ows[nxt] is not None:
                    ows[nxt].wait()
                    ows[nxt] = None
                start_in(c + 1)
            grs[buf].wait()
            frs[buf].wait()

            rows_ref = rows[buf]
            featb_ref = featb[buf]

            @plsc.parallel_loop(0, chunk, 1, unroll=4)
            def _row(j):
                for kk in range(vecs_per_row):
                    sl = pl.ds(kk * _LANES, _LANES)
                    plsc.addupdate(featb_ref.at[j, sl], rows_ref[j, sl])

            ows[buf] = pltpu.async_copy(
                featb[buf], out_hbm.at[pl.ds(base + c * chunk, chunk)], osem[buf])
        for d in ows:
            if d is not None:
                d.wait()

    k(W, feat, deg, out_r)


_SPLIT = 7680


def kernel(x, feat, W):
    n = x.shape[1]
    vmax = W.shape[0] - 1
    out_r = jax.new_ref(jnp.zeros(feat.shape, feat.dtype))
    deg_a = _degree(x, vmax, 0, _SPLIT, block_cols=_SPLIT)
    _gather_add(out_r, W, feat, deg_a, 0, _SPLIT, b_per_w=_SPLIT // 32)
    deg_b = _degree(x, vmax, _SPLIT, n - _SPLIT)
    _gather_add(out_r, W, feat, deg_b, _SPLIT, n - _SPLIT, b_per_w=80)
    return out_r[...]


# speedup vs baseline: 1.4337x; 1.1304x over previous
import functools

import jax
import jax.numpy as jnp
from jax import lax
from jax.experimental import pallas as pl
from jax.experimental.pallas import tpu as pltpu
from jax.experimental.pallas import tpu_sc as plsc


def _degree_body(vmax, x_ref, deg_ref, acc_ref):
    i = pl.program_id(0)

    @pl.when(i == 0)
    def _init():
        acc_ref[...] = jnp.zeros_like(acc_ref)

    acc_ref[...] += jnp.sum(x_ref[...], axis=0, keepdims=True)

    @pl.when(i == pl.num_programs(0) - 1)
    def _fin():
        deg = jnp.ceil(acc_ref[...]).astype(jnp.int32)
        deg_ref[...] = jnp.clip(deg, 0, vmax)


def _degree(x, vmax, block_rows=200):
    n_rows, n_cols = x.shape
    grid = (pl.cdiv(n_rows, block_rows),)
    out = pl.pallas_call(
        functools.partial(_degree_body, vmax),
        grid=grid,
        in_specs=[pl.BlockSpec((block_rows, n_cols), lambda i: (i, 0))],
        out_specs=pl.BlockSpec((1, n_cols), lambda i: (0, 0)),
        out_shape=jax.ShapeDtypeStruct((1, n_cols), jnp.int32),
        scratch_shapes=[pltpu.VMEM((1, n_cols), jnp.float32)],
    )(x)
    return out.reshape(n_cols)


_LANES = 16
_NBUF = 3


def _gather_add(W, feat, deg, b_per_w=320, chunk=80):
    B, D = feat.shape
    mesh = plsc.VectorSubcoreMesh(core_axis_name="c", subcore_axis_name="s")
    n_chunks = b_per_w // chunk
    vecs_per_row = D // _LANES
    nbuf = min(_NBUF, n_chunks)

    @functools.partial(
        pl.kernel,
        mesh=mesh,
        out_type=jax.ShapeDtypeStruct((B, D), jnp.float32),
        scratch_types=[
            pltpu.VMEM((b_per_w,), jnp.int32),
            [pltpu.VMEM((chunk, D), jnp.float32) for _ in range(nbuf)],
            [pltpu.VMEM((chunk, D), jnp.float32) for _ in range(nbuf)],
            [pltpu.SemaphoreType.DMA for _ in range(nbuf)],
            [pltpu.SemaphoreType.DMA for _ in range(nbuf)],
            [pltpu.SemaphoreType.DMA for _ in range(nbuf)],
        ],
    )
    def k(w_hbm, feat_hbm, deg_hbm, out_hbm,
          idx_v, rows, featb, gsem, fsem, osem):
        wid = lax.axis_index("s") * 2 + lax.axis_index("c")
        base = jnp.minimum(wid * b_per_w, B - b_per_w)

        grs = [None] * nbuf
        frs = [None] * nbuf
        ows = [None] * nbuf

        def start_feat(c):
            buf = c % nbuf
            frs[buf] = pltpu.async_copy(
                feat_hbm.at[pl.ds(base + c * chunk, chunk)], featb[buf], fsem[buf])

        def start_gather(c):
            buf = c % nbuf
            grs[buf] = pltpu.async_copy(
                w_hbm.at[idx_v.at[pl.ds(c * chunk, chunk)]], rows[buf], gsem[buf])

        n_prime = min(nbuf - 1, n_chunks) if nbuf > 1 else 1
        for c in range(n_prime):
            start_feat(c)
        pltpu.sync_copy(deg_hbm.at[pl.ds(base, b_per_w)], idx_v)
        for c in range(n_prime):
            start_gather(c)

        for c in range(n_chunks):
            buf = c % nbuf
            nc = c + nbuf - 1
            if nbuf > 1 and nc < n_chunks:
                nbuf_i = nc % nbuf
                if ows[nbuf_i] is not None:
                    ows[nbuf_i].wait()
                    ows[nbuf_i] = None
                start_gather(nc)
                start_feat(nc)
            grs[buf].wait()
            frs[buf].wait()

            rows_ref = rows[buf]
            featb_ref = featb[buf]

            @plsc.parallel_loop(0, chunk, 1, unroll=4)
            def _row(j):
                for kk in range(vecs_per_row):
                    sl = pl.ds(kk * _LANES, _LANES)
                    plsc.addupdate(featb_ref.at[j, sl], rows_ref[j, sl])

            ows[buf] = pltpu.async_copy(
                featb[buf], out_hbm.at[pl.ds(base + c * chunk, chunk)], osem[buf])
        for d in ows:
            if d is not None:
                d.wait()

    return k(W, feat, deg)


def kernel(x, feat, W):
    deg = _degree(x, W.shape[0] - 1)
    return _gather_add(W, feat, deg)
